# baseline (speedup 1.0000x reference)
import functools

import jax
import jax.numpy as jnp
from jax import lax
from jax.experimental import pallas as pl
from jax.experimental.pallas import tpu as pltpu
from jax.experimental.pallas import tpu_sc as plsc

_N = 10000
_E = 320000
_D = 128
_NS = 16
_CS = 40
_NP = 10240
_RPT = _NP // _NS

_f32 = jnp.float32


def _dot(a, b):
    return jnp.dot(a, b, preferred_element_type=_f32)


def _input_proj(h, p, Wh, bh, Wp, bp):
    def body(h_ref, p_ref, wh_ref, bh_ref, wp_ref, bp_ref, o_ref):
        o_ref[...] = (_dot(h_ref[...], wh_ref[...]) + bh_ref[...]
                      + _dot(p_ref[...], wp_ref[...]) + bp_ref[...])

    blk = 1000
    return pl.pallas_call(
        body,
        grid=(_N // blk,),
        in_specs=[
            pl.BlockSpec((blk, _D), lambda i: (i, 0)),
            pl.BlockSpec((blk, 16), lambda i: (i, 0)),
            pl.BlockSpec((_D, _D), lambda i: (0, 0)),
            pl.BlockSpec((1, _D), lambda i: (0, 0)),
            pl.BlockSpec((16, _D), lambda i: (0, 0)),
            pl.BlockSpec((1, _D), lambda i: (0, 0)),
        ],
        out_specs=pl.BlockSpec((blk, _D), lambda i: (i, 0)),
        out_shape=jax.ShapeDtypeStruct((_N, _D), _f32),
    )(h, p, Wh, bh.reshape(1, _D), Wp, bp.reshape(1, _D))


def _node_proj(h, WA, bA, WB, bB, WD, bD, WE, bE):
    def body(h_ref, wa, ba, wb, bb, wd, bd, we, be, a_ref, b_ref, d_ref, e_ref):
        x = h_ref[...]
        a_ref[...] = _dot(x, wa[...]) + ba[...]
        b_ref[...] = _dot(x, wb[...]) + bb[...]
        d_ref[...] = _dot(x, wd[...]) + bd[...]
        e_ref[...] = _dot(x, we[...]) + be[...]

    blk = 1000
    wspec = pl.BlockSpec((_D, _D), lambda i: (0, 0))
    bspec = pl.BlockSpec((1, _D), lambda i: (0, 0))
    nspec = pl.BlockSpec((blk, _D), lambda i: (i, 0))
    return pl.pallas_call(
        body,
        grid=(_N // blk,),
        in_specs=[nspec, wspec, bspec, wspec, bspec, wspec, bspec, wspec,
                  bspec],
        out_specs=[nspec, nspec, nspec, nspec],
        out_shape=[jax.ShapeDtypeStruct((_N, _D), _f32)] * 4,
    )(h, WA, bA.reshape(1, _D), WB, bB.reshape(1, _D),
      WD, bD.reshape(1, _D), WE, bE.reshape(1, _D))


def _edge_first(e, We, be, WC, bC):
    def body(e_ref, we, be_, wc, bc, e0_ref, ce_ref):
        x = _dot(e_ref[...], we[...]) + be_[...]
        e0_ref[...] = x
        ce_ref[...] = _dot(x, wc[...]) + bc[...]

    blk = 2560
    wspec = pl.BlockSpec((_D, _D), lambda i: (0, 0))
    bspec = pl.BlockSpec((1, _D), lambda i: (0, 0))
    espec = pl.BlockSpec((blk, _D), lambda i: (i, 0))
    return pl.pallas_call(
        body,
        grid=(_E // blk,),
        in_specs=[espec, wspec, bspec, wspec, bspec],
        out_specs=[espec, espec],
        out_shape=[jax.ShapeDtypeStruct((_E, _D), _f32)] * 2,
    )(e, We, be.reshape(1, _D), WC, bC.reshape(1, _D))


def _edge_apply(e_in, enew, parts, ge_l, bte_l, WC, bC):
    def body(ein_ref, en_ref, part_ref, g_ref, b_ref, wc, bc, eo_ref, ce_ref):
        p = jnp.sum(part_ref[...], axis=0)
        mean = p[0] * (1.0 / _E)
        var = p[1] * (1.0 / _E) - mean * mean
        scale = g_ref[...] / jnp.sqrt(var + 1e-5)
        xb = scale * (en_ref[...] - mean) + b_ref[...]
        eo = ein_ref[...] + jnp.maximum(xb, 0.0)
        eo_ref[...] = eo
        ce_ref[...] = _dot(eo, wc[...]) + bc[...]

    blk = 2560
    espec = pl.BlockSpec((blk, _D), lambda i: (i, 0))
    return pl.pallas_call(
        body,
        grid=(_E // blk,),
        in_specs=[
            espec,
            espec,
            pl.BlockSpec((_NS, 2, _D), lambda i: (0, 0, 0)),
            pl.BlockSpec((1, _D), lambda i: (0, 0)),
            pl.BlockSpec((1, _D), lambda i: (0, 0)),
            pl.BlockSpec((_D, _D), lambda i: (0, 0)),
            pl.BlockSpec((1, _D), lambda i: (0, 0)),
        ],
        out_specs=[espec, espec],
        out_shape=[jax.ShapeDtypeStruct((_E, _D), _f32)] * 2,
    )(e_in, enew, parts, ge_l.reshape(1, _D), bte_l.reshape(1, _D),
      WC, bC.reshape(1, _D))


def _h_update(Ah, ss, ssh, h_in, gh_l, bth_l):
    def body(ah_ref, ss_ref, ssh_ref, hin_ref, g_ref, b_ref, ho_ref):
        hn = ah_ref[...] + ssh_ref[...] / (ss_ref[...] + 1e-6)
        m = jnp.mean(hn, axis=0)
        v = jnp.mean((hn - m) ** 2, axis=0)
        hb = g_ref[...] * (hn - m) / jnp.sqrt(v + 1e-5) + b_ref[...]
        ho_ref[...] = hin_ref[...] + jnp.maximum(hb, 0.0)

    nspec = pl.BlockSpec((_N, _D), lambda i: (0, 0))
    return pl.pallas_call(
        body,
        grid=(1,),
        in_specs=[nspec, nspec, nspec, nspec,
                  pl.BlockSpec((1, _D), lambda i: (0, 0)),
                  pl.BlockSpec((1, _D), lambda i: (0, 0))],
        out_specs=nspec,
        out_shape=jax.ShapeDtypeStruct((_N, _D), _f32),
    )(Ah, ss, ssh, h_in, gh_l.reshape(1, _D), bth_l.reshape(1, _D))


def _readout(h, W1, b1, W2, b2, W3, b3):
    def body(h_ref, w1, b1_, w2, b2_, w3, b3_, y_ref):
        hg = jnp.mean(h_ref[...], axis=0, keepdims=True)
        y = jnp.maximum(_dot(hg, w1[...]) + b1_[...], 0.0)
        y = jnp.maximum(_dot(y, w2[...]) + b2_[...], 0.0)
        y_ref[...] = _dot(y, w3[...]) + b3_[...]

    return pl.pallas_call(
        body,
        grid=(1,),
        in_specs=[pl.BlockSpec(h.shape, lambda i: (0, 0)),
                  pl.BlockSpec(W1.shape, lambda i: (0, 0)),
                  pl.BlockSpec((1, 64), lambda i: (0, 0)),
                  pl.BlockSpec(W2.shape, lambda i: (0, 0)),
                  pl.BlockSpec((1, 32), lambda i: (0, 0)),
                  pl.BlockSpec(W3.shape, lambda i: (0, 0)),
                  pl.BlockSpec((1, 10), lambda i: (0, 0))],
        out_specs=pl.BlockSpec((1, 10), lambda i: (0, 0)),
        out_shape=jax.ShapeDtypeStruct((1, 10), _f32),
    )(h, W1, b1.reshape(1, 64), W2, b2.reshape(1, 32), W3, b3.reshape(1, 10))


def _sc_pass_ss(ce, dh, eh, src, dst, write_enew=True):
    per_tile = _E // _NS
    chunks = per_tile // _CS
    mesh = plsc.VectorSubcoreMesh(core_axis_name="c", subcore_axis_name="s",
                                  num_cores=1)

    enew_shape = (_E, _D) if write_enew else (8, _D)
    out_type = [
        jax.ShapeDtypeStruct(enew_shape, _f32),
        jax.ShapeDtypeStruct((_E, _D), _f32),
        jax.ShapeDtypeStruct((_NP, _D), _f32),
        jax.ShapeDtypeStruct((_NS, 2, _D), _f32),
    ]
    scratch_types = (
        [pltpu.VMEM((_CS,), jnp.int32)] * 8
        + [pltpu.VMEM((_CS, _D), _f32)] * 8
        + [pltpu.VMEM((2, _D), _f32)]
        + [pltpu.VMEM_SHARED((_NP, _D), _f32)]
        + [pltpu.SemaphoreType.DMA] * 16
    )

    @functools.partial(pl.kernel, out_type=out_type, mesh=mesh,
                       scratch_types=scratch_types)
    def k(ce_hbm, dh_hbm, eh_hbm, src_hbm, dst_hbm,
          enew_hbm, sg_hbm, acc_hbm, stats_hbm,
          si0, si1, si2, si3, di0, di1, di2, di3,
          a_v0, a_v1, b_v0, b_v1, c_v0, c_v1, d_v0, d_v1,
          stats_b, acc_sh, *sems):
        s = lax.axis_index("s")
        src_i = (si0, si1, si2, si3)
        dst_i = (di0, di1, di2, di3)
        a_v = (a_v0, a_v1)
        b_v = (b_v0, b_v1)
        c_v = (c_v0, c_v1)
        d_v = (d_v0, d_v1)
        sm_ix = sems[0:4]
        smg_a = sems[4:6]
        smg_b = sems[6:8]
        smg_c = sems[8:10]
        smw_en = sems[10:12]
        smw_sg = sems[12:14]
        smw_sc = sems[14:16]

        def zrow(j, carry):
            for kk in range(_D // 16):
                d_v0[j, pl.ds(kk * 16, 16)] = jnp.zeros((16,), _f32)
            return carry
        lax.fori_loop(0, _CS, zrow, 0)
        for j in range(_RPT // _CS):
            off = s * _RPT + j * _CS
            pltpu.sync_copy(d_v0, acc_sh.at[pl.ds(off, _CS), :])
        plsc.subcore_barrier()
        for kk in range(_D // 16):
            sl = pl.ds(kk * 16, 16)
            stats_b[0, sl] = jnp.zeros((16,), _f32)
            stats_b[1, sl] = jnp.zeros((16,), _f32)

        tbase = s * per_tile

        def fire_idx(j, q):
            @pl.when(j < chunks)
            def _():
                base = tbase + j * _CS
                pltpu.async_copy(src_hbm.at[pl.ds(base, _CS)], src_i[q],
                                 sm_ix[q])
                pltpu.async_copy(dst_hbm.at[pl.ds(base, _CS)], dst_i[q],
                                 sm_ix[q])

        def wait_idx(j, q):
            @pl.when(j < chunks)
            def _():
                pltpu.make_async_copy(src_hbm.at[pl.ds(tbase, _CS)],
                                      src_i[q], sm_ix[q]).wait()
                pltpu.make_async_copy(dst_hbm.at[pl.ds(tbase, _CS)],
                                      dst_i[q], sm_ix[q]).wait()

        def fire_gathers(j, q, b):
            @pl.when(j < chunks)
            def _():
                base = tbase + j * _CS
                pltpu.async_copy(dh_hbm.at[src_i[q]], a_v[b], smg_a[b])
                pltpu.async_copy(eh_hbm.at[dst_i[q]], b_v[b], smg_b[b])
                pltpu.async_copy(ce_hbm.at[pl.ds(base, _CS), :], c_v[b],
                                 smg_c[b])

        def wait_writes(j, b, q):
            @pl.when(j >= 0)
            def _():
                if write_enew:
                    pltpu.make_async_copy(
                        c_v[b], enew_hbm.at[pl.ds(tbase, _CS), :],
                        smw_en[b]).wait()
                pltpu.make_async_copy(
                    d_v[b], sg_hbm.at[pl.ds(tbase, _CS), :], smw_sg[b]).wait()
                pltpu.make_async_copy(
                    d_v[b], acc_sh.at[dst_i[q]], smw_sc[b]).wait()

        def work(j, q, b):
            base = tbase + j * _CS
            pltpu.make_async_copy(dh_hbm.at[src_i[q]], a_v[b],
                                  smg_a[b]).wait()
            pltpu.make_async_copy(eh_hbm.at[dst_i[q]], b_v[b],
                                  smg_b[b]).wait()
            pltpu.make_async_copy(ce_hbm.at[pl.ds(base, _CS), :], c_v[b],
                                  smg_c[b]).wait()
            z = jnp.zeros((16,), _f32)

            @plsc.parallel_loop(0, _CS, unroll=4, carry=(z,) * 16)
            def cr(jj, st):
                st = list(st)
                for kk in range(_D // 16):
                    sl = pl.ds(kk * 16, 16)
                    en = c_v[b][jj, sl] + a_v[b][jj, sl] + b_v[b][jj, sl]
                    sg = 1.0 / (1.0 + jnp.exp(-en))
                    c_v[b][jj, sl] = en
                    d_v[b][jj, sl] = sg
                    st[kk] = st[kk] + en
                    st[8 + kk] = st[8 + kk] + en * en
                return tuple(st)

            for kk in range(_D // 16):
                sl = pl.ds(kk * 16, 16)
                stats_b[0, sl] = stats_b[0, sl] + cr[kk]
                stats_b[1, sl] = stats_b[1, sl] + cr[8 + kk]
            if write_enew:
                pltpu.async_copy(c_v[b], enew_hbm.at[pl.ds(base, _CS), :],
                                 smw_en[b])
            pltpu.async_copy(d_v[b], sg_hbm.at[pl.ds(base, _CS), :],
                             smw_sg[b])
            pltpu.async_copy(d_v[b], acc_sh.at[dst_i[q]], smw_sc[b],
                             add=True)

        fire_idx(0, 0)
        fire_idx(1, 1)
        wait_idx(0, 0)
        fire_gathers(0, 0, 0)

        def group(i, carry):
            for q in range(4):
                jv = i * 4 + q
                b = q % 2
                fire_idx(jv + 2, (q + 2) % 4)
                wait_writes(jv - 1, 1 - b, (q + 3) % 4)
                wait_idx(jv + 1, (q + 1) % 4)
                fire_gathers(jv + 1, (q + 1) % 4, 1 - b)
                work(jv, q, b)
            return carry

        lax.fori_loop(0, chunks // 4, group, 0)
        wait_writes(chunks - 1, 1, 3)

        pltpu.sync_copy(stats_b, stats_hbm.at[s])

        plsc.subcore_barrier()
        rbase = s * _RPT
        pltpu.sync_copy(acc_sh.at[pl.ds(rbase, _RPT), :],
                        acc_hbm.at[pl.ds(rbase, _RPT), :])

    return k(ce, dh, eh, src, dst)


def _sc_pass_ssh(sg, bh, src, dst):
    per_tile = _E // _NS
    chunks = per_tile // _CS
    mesh = plsc.VectorSubcoreMesh(core_axis_name="c", subcore_axis_name="s",
                                  num_cores=1)

    out_type = [jax.ShapeDtypeStruct((_NP, _D), _f32)]
    scratch_types = (
        [pltpu.VMEM((_CS,), jnp.int32)] * 8
        + [pltpu.VMEM((_CS, _D), _f32)] * 4
        + [pltpu.VMEM_SHARED((_NP, _D), _f32)]
        + [pltpu.SemaphoreType.DMA] * 10
    )

    @functools.partial(pl.kernel, out_type=out_type, mesh=mesh,
                       scratch_types=scratch_types)
    def k(sg_hbm, bh_hbm, src_hbm, dst_hbm, acc_hbm,
          si0, si1, si2, si3, di0, di1, di2, di3,
          sg_v0, sg_v1, bh_v0, bh_v1, acc_sh, *sems):
        s = lax.axis_index("s")
        src_i = (si0, si1, si2, si3)
        dst_i = (di0, di1, di2, di3)
        sg_v = (sg_v0, sg_v1)
        bh_v = (bh_v0, bh_v1)
        sm_ix = sems[0:4]
        smg = sems[4:6]
        sml = sems[6:8]
        smw = sems[8:10]

        def zrow(j, carry):
            for kk in range(_D // 16):
                sg_v0[j, pl.ds(kk * 16, 16)] = jnp.zeros((16,), _f32)
            return carry
        lax.fori_loop(0, _CS, zrow, 0)
        for j in range(_RPT // _CS):
            off = s * _RPT + j * _CS
            pltpu.sync_copy(sg_v0, acc_sh.at[pl.ds(off, _CS), :])
        plsc.subcore_barrier()

        tbase = s * per_tile

        def fire_idx(j, q):
            @pl.when(j < chunks)
            def _():
                base = tbase + j * _CS
                pltpu.async_copy(src_hbm.at[pl.ds(base, _CS)], src_i[q],
                                 sm_ix[q])
                pltpu.async_copy(dst_hbm.at[pl.ds(base, _CS)], dst_i[q],
                                 sm_ix[q])

        def wait_idx(j, q):
            @pl.when(j < chunks)
            def _():
                pltpu.make_async_copy(src_hbm.at[pl.ds(tbase, _CS)],
                                      src_i[q], sm_ix[q]).wait()
                pltpu.make_async_copy(dst_hbm.at[pl.ds(tbase, _CS)],
                                      dst_i[q], sm_ix[q]).wait()

        def fire_gathers(j, q, b):
            @pl.when(j < chunks)
            def _():
                base = tbase + j * _CS
                pltpu.async_copy(bh_hbm.at[src_i[q]], bh_v[b], smg[b])
                pltpu.async_copy(sg_hbm.at[pl.ds(base, _CS), :], sg_v[b],
                                 sml[b])

        def wait_writes(j, b, q):
            @pl.when(j >= 0)
            def _():
                pltpu.make_async_copy(sg_v[b], acc_sh.at[dst_i[q]],
                                      smw[b]).wait()

        def work(j, q, b):
            base = tbase + j * _CS
            pltpu.make_async_copy(bh_hbm.at[src_i[q]], bh_v[b], smg[b]).wait()
            pltpu.make_async_copy(sg_hbm.at[pl.ds(base, _CS), :], sg_v[b],
                                  sml[b]).wait()

            @plsc.parallel_loop(0, _CS, unroll=8)
            def _(jj):
                for kk in range(_D // 16):
                    sl = pl.ds(kk * 16, 16)
                    sg_v[b][jj, sl] = sg_v[b][jj, sl] * bh_v[b][jj, sl]

            pltpu.async_copy(sg_v[b], acc_sh.at[dst_i[q]], smw[b], add=True)

        fire_idx(0, 0)
        fire_idx(1, 1)
        wait_idx(0, 0)
        fire_gathers(0, 0, 0)

        def group(i, carry):
            for q in range(4):
                jv = i * 4 + q
                b = q % 2
                fire_idx(jv + 2, (q + 2) % 4)
                wait_writes(jv - 1, 1 - b, (q + 3) % 4)
                wait_idx(jv + 1, (q + 1) % 4)
                fire_gathers(jv + 1, (q + 1) % 4, 1 - b)
                work(jv, q, b)
            return carry

        lax.fori_loop(0, chunks // 4, group, 0)
        wait_writes(chunks - 1, 1, 3)

        plsc.subcore_barrier()
        rbase = s * _RPT
        pltpu.sync_copy(acc_sh.at[pl.ds(rbase, _RPT), :],
                        acc_hbm.at[pl.ds(rbase, _RPT), :])

    return k(sg, bh, src, dst)


def kernel(h, p, e, snorm_n, edge_index, Wh, bh, We, be, Wp, bp, WA, bA, WB,
           bB, WC, bC, WD, bD, WEm, bEm, gh, bth, ge, bte, W1, b1, W2, b2,
           W3, b3):
    del snorm_n
    src = edge_index[0]
    dst = edge_index[1]
    L = WA.shape[0]

    hs = _input_proj(h, p, Wh, bh, Wp, bp)
    e_in = None
    enew_prev = None
    parts_prev = None
    for l in range(L):
        if l == 0:
            e_in, ce = _edge_first(e, We, be, WC[0], bC[0])
        else:
            e_in, ce = _edge_apply(e_in, enew_prev, parts_prev,
                                   ge[l - 1], bte[l - 1], WC[l], bC[l])
        Ah, Bh, Dh, Eh = _node_proj(hs, WA[l], bA[l], WB[l], bB[l],
                                    WD[l], bD[l], WEm[l], bEm[l])
        last = l == L - 1
        enew_prev, sg, ss, parts_prev = _sc_pass_ss(ce, Dh, Eh, src, dst,
                                                    write_enew=not last)
        (ssh,) = _sc_pass_ssh(sg, Bh, src, dst)
        hs = _h_update(Ah, ss[:_N], ssh[:_N], hs, gh[l], bth[l])

    y = _readout(hs, W1, b1, W2, b2, W3, b3)
    return y.reshape(10)


# speedup vs baseline: 1.0068x; 1.0068x over previous
import functools

import jax
import jax.numpy as jnp
from jax import lax
from jax.experimental import pallas as pl
from jax.experimental.pallas import tpu as pltpu
from jax.experimental.pallas import tpu_sc as plsc

_N = 10000
_E = 320000
_D = 128
_NS = 16
_CS = 40
_NP = 10240
_RPT = _NP // _NS

_f32 = jnp.float32


def _dot(a, b):
    return jnp.dot(a, b, preferred_element_type=_f32)


def _input_proj(h, p, Wh, bh, Wp, bp):
    def body(h_ref, p_ref, wh_ref, bh_ref, wp_ref, bp_ref, o_ref):
        o_ref[...] = (_dot(h_ref[...], wh_ref[...]) + bh_ref[...]
                      + _dot(p_ref[...], wp_ref[...]) + bp_ref[...])

    blk = 1000
    return pl.pallas_call(
        body,
        grid=(_N // blk,),
        in_specs=[
            pl.BlockSpec((blk, _D), lambda i: (i, 0)),
            pl.BlockSpec((blk, 16), lambda i: (i, 0)),
            pl.BlockSpec((_D, _D), lambda i: (0, 0)),
            pl.BlockSpec((1, _D), lambda i: (0, 0)),
            pl.BlockSpec((16, _D), lambda i: (0, 0)),
            pl.BlockSpec((1, _D), lambda i: (0, 0)),
        ],
        out_specs=pl.BlockSpec((blk, _D), lambda i: (i, 0)),
        out_shape=jax.ShapeDtypeStruct((_N, _D), _f32),
    )(h, p, Wh, bh.reshape(1, _D), Wp, bp.reshape(1, _D))


def _node_proj(h, WA, bA, WB, bB, WD, bD, WE, bE):
    def body(h_ref, wa, ba, wb, bb, wd, bd, we, be, a_ref, b_ref, d_ref, e_ref):
        x = h_ref[...]
        a_ref[...] = _dot(x, wa[...]) + ba[...]
        b_ref[...] = _dot(x, wb[...]) + bb[...]
        d_ref[...] = _dot(x, wd[...]) + bd[...]
        e_ref[...] = _dot(x, we[...]) + be[...]

    blk = 1000
    wspec = pl.BlockSpec((_D, _D), lambda i: (0, 0))
    bspec = pl.BlockSpec((1, _D), lambda i: (0, 0))
    nspec = pl.BlockSpec((blk, _D), lambda i: (i, 0))
    return pl.pallas_call(
        body,
        grid=(_N // blk,),
        in_specs=[nspec, wspec, bspec, wspec, bspec, wspec, bspec, wspec,
                  bspec],
        out_specs=[nspec, nspec, nspec, nspec],
        out_shape=[jax.ShapeDtypeStruct((_N, _D), _f32)] * 4,
    )(h, WA, bA.reshape(1, _D), WB, bB.reshape(1, _D),
      WD, bD.reshape(1, _D), WE, bE.reshape(1, _D))


def _edge_first(e, We, be, WC, bC):
    def body(e_ref, we, be_, wc, bc, e0_ref, ce_ref):
        x = _dot(e_ref[...], we[...]) + be_[...]
        e0_ref[...] = x
        ce_ref[...] = _dot(x, wc[...]) + bc[...]

    blk = 2560
    wspec = pl.BlockSpec((_D, _D), lambda i: (0, 0))
    bspec = pl.BlockSpec((1, _D), lambda i: (0, 0))
    espec = pl.BlockSpec((blk, _D), lambda i: (i, 0))
    return pl.pallas_call(
        body,
        grid=(_E // blk,),
        in_specs=[espec, wspec, bspec, wspec, bspec],
        out_specs=[espec, espec],
        out_shape=[jax.ShapeDtypeStruct((_E, _D), _f32)] * 2,
    )(e, We, be.reshape(1, _D), WC, bC.reshape(1, _D))


def _edge_apply(e_in, enew, parts, ge_l, bte_l, WC, bC):
    def body(ein_ref, en_ref, part_ref, g_ref, b_ref, wc, bc, eo_ref, ce_ref):
        p = jnp.sum(part_ref[...], axis=0)
        mean = p[0] * (1.0 / _E)
        var = p[1] * (1.0 / _E) - mean * mean
        scale = g_ref[...] / jnp.sqrt(var + 1e-5)
        xb = scale * (en_ref[...] - mean) + b_ref[...]
        eo = ein_ref[...] + jnp.maximum(xb, 0.0)
        eo_ref[...] = eo
        ce_ref[...] = _dot(eo, wc[...]) + bc[...]

    blk = 2560
    espec = pl.BlockSpec((blk, _D), lambda i: (i, 0))
    return pl.pallas_call(
        body,
        grid=(_E // blk,),
        in_specs=[
            espec,
            espec,
            pl.BlockSpec((_NS, 2, _D), lambda i: (0, 0, 0)),
            pl.BlockSpec((1, _D), lambda i: (0, 0)),
            pl.BlockSpec((1, _D), lambda i: (0, 0)),
            pl.BlockSpec((_D, _D), lambda i: (0, 0)),
            pl.BlockSpec((1, _D), lambda i: (0, 0)),
        ],
        out_specs=[espec, espec],
        out_shape=[jax.ShapeDtypeStruct((_E, _D), _f32)] * 2,
    )(e_in, enew, parts, ge_l.reshape(1, _D), bte_l.reshape(1, _D),
      WC, bC.reshape(1, _D))


def _h_update(Ah, ss, ssh, h_in, gh_l, bth_l):
    def body(ah_ref, ss_ref, ssh_ref, hin_ref, g_ref, b_ref, ho_ref):
        hn = ah_ref[...] + ssh_ref[...] / (ss_ref[...] + 1e-6)
        m = jnp.mean(hn, axis=0)
        v = jnp.mean((hn - m) ** 2, axis=0)
        hb = g_ref[...] * (hn - m) / jnp.sqrt(v + 1e-5) + b_ref[...]
        ho_ref[...] = hin_ref[...] + jnp.maximum(hb, 0.0)

    nspec = pl.BlockSpec((_N, _D), lambda i: (0, 0))
    return pl.pallas_call(
        body,
        grid=(1,),
        in_specs=[nspec, nspec, nspec, nspec,
                  pl.BlockSpec((1, _D), lambda i: (0, 0)),
                  pl.BlockSpec((1, _D), lambda i: (0, 0))],
        out_specs=nspec,
        out_shape=jax.ShapeDtypeStruct((_N, _D), _f32),
    )(Ah, ss, ssh, h_in, gh_l.reshape(1, _D), bth_l.reshape(1, _D))


def _readout(h, W1, b1, W2, b2, W3, b3):
    def body(h_ref, w1, b1_, w2, b2_, w3, b3_, y_ref):
        hg = jnp.mean(h_ref[...], axis=0, keepdims=True)
        y = jnp.maximum(_dot(hg, w1[...]) + b1_[...], 0.0)
        y = jnp.maximum(_dot(y, w2[...]) + b2_[...], 0.0)
        y_ref[...] = _dot(y, w3[...]) + b3_[...]

    return pl.pallas_call(
        body,
        grid=(1,),
        in_specs=[pl.BlockSpec(h.shape, lambda i: (0, 0)),
                  pl.BlockSpec(W1.shape, lambda i: (0, 0)),
                  pl.BlockSpec((1, 64), lambda i: (0, 0)),
                  pl.BlockSpec(W2.shape, lambda i: (0, 0)),
                  pl.BlockSpec((1, 32), lambda i: (0, 0)),
                  pl.BlockSpec(W3.shape, lambda i: (0, 0)),
                  pl.BlockSpec((1, 10), lambda i: (0, 0))],
        out_specs=pl.BlockSpec((1, 10), lambda i: (0, 0)),
        out_shape=jax.ShapeDtypeStruct((1, 10), _f32),
    )(h, W1, b1.reshape(1, 64), W2, b2.reshape(1, 32), W3, b3.reshape(1, 10))


def _sc_pass_ss(ce, dh, eh, src, dst, write_enew=True):
    per_tile = _E // _NS
    chunks = per_tile // _CS
    mesh = plsc.VectorSubcoreMesh(core_axis_name="c", subcore_axis_name="s",
                                  num_cores=1)

    enew_shape = (_E, _D) if write_enew else (8, _D)
    out_type = [
        jax.ShapeDtypeStruct(enew_shape, _f32),
        jax.ShapeDtypeStruct((_E, _D), _f32),
        jax.ShapeDtypeStruct((_NP, _D), _f32),
        jax.ShapeDtypeStruct((_NS, 2, _D), _f32),
    ]
    scratch_types = (
        [pltpu.VMEM((_CS,), jnp.int32)] * 8
        + [pltpu.VMEM((_CS, _D), _f32)] * 8
        + [pltpu.VMEM((2, _D), _f32)]
        + [pltpu.VMEM_SHARED((_NP, _D), _f32)]
        + [pltpu.SemaphoreType.DMA] * 16
    )

    @functools.partial(pl.kernel, out_type=out_type, mesh=mesh,
                       scratch_types=scratch_types)
    def k(ce_hbm, dh_hbm, eh_hbm, src_hbm, dst_hbm,
          enew_hbm, sg_hbm, acc_hbm, stats_hbm,
          si0, si1, si2, si3, di0, di1, di2, di3,
          a_v0, a_v1, b_v0, b_v1, c_v0, c_v1, d_v0, d_v1,
          stats_b, acc_sh, *sems):
        s = lax.axis_index("s")
        src_i = (si0, si1, si2, si3)
        dst_i = (di0, di1, di2, di3)
        a_v = (a_v0, a_v1)
        b_v = (b_v0, b_v1)
        c_v = (c_v0, c_v1)
        d_v = (d_v0, d_v1)
        sm_ix = sems[0:4]
        smg_a = sems[4:6]
        smg_b = sems[6:8]
        smg_c = sems[8:10]
        smw_en = sems[10:12]
        smw_sg = sems[12:14]
        smw_sc = sems[14:16]

        def zrow(j, carry):
            for kk in range(_D // 16):
                d_v0[j, pl.ds(kk * 16, 16)] = jnp.zeros((16,), _f32)
            return carry
        lax.fori_loop(0, _CS, zrow, 0)
        for j in range(_RPT // _CS):
            off = s * _RPT + j * _CS
            pltpu.sync_copy(d_v0, acc_sh.at[pl.ds(off, _CS), :])
        plsc.subcore_barrier()
        for kk in range(_D // 16):
            sl = pl.ds(kk * 16, 16)
            stats_b[0, sl] = jnp.zeros((16,), _f32)
            stats_b[1, sl] = jnp.zeros((16,), _f32)

        tbase = s * per_tile

        def fire_idx(j, q):
            @pl.when(j < chunks)
            def _():
                base = tbase + j * _CS
                pltpu.async_copy(src_hbm.at[pl.ds(base, _CS)], src_i[q],
                                 sm_ix[q])
                pltpu.async_copy(dst_hbm.at[pl.ds(base, _CS)], dst_i[q],
                                 sm_ix[q])

        def wait_idx(j, q):
            @pl.when(j < chunks)
            def _():
                pltpu.make_async_copy(src_hbm.at[pl.ds(tbase, _CS)],
                                      src_i[q], sm_ix[q]).wait()
                pltpu.make_async_copy(dst_hbm.at[pl.ds(tbase, _CS)],
                                      dst_i[q], sm_ix[q]).wait()

        def fire_gathers(j, q, b):
            @pl.when(j < chunks)
            def _():
                base = tbase + j * _CS
                pltpu.async_copy(dh_hbm.at[src_i[q]], a_v[b], smg_a[b])
                pltpu.async_copy(eh_hbm.at[dst_i[q]], b_v[b], smg_b[b])
                pltpu.async_copy(ce_hbm.at[pl.ds(base, _CS), :], c_v[b],
                                 smg_c[b])

        def wait_writes(j, b, q):
            @pl.when(j >= 0)
            def _():
                if write_enew:
                    pltpu.make_async_copy(
                        c_v[b], enew_hbm.at[pl.ds(tbase, _CS), :],
                        smw_en[b]).wait()
                pltpu.make_async_copy(
                    d_v[b], sg_hbm.at[pl.ds(tbase, _CS), :], smw_sg[b]).wait()
                pltpu.make_async_copy(
                    d_v[b], acc_sh.at[dst_i[q]], smw_sc[b]).wait()

        def work(j, q, b):
            base = tbase + j * _CS
            pltpu.make_async_copy(dh_hbm.at[src_i[q]], a_v[b],
                                  smg_a[b]).wait()
            pltpu.make_async_copy(eh_hbm.at[dst_i[q]], b_v[b],
                                  smg_b[b]).wait()
            pltpu.make_async_copy(ce_hbm.at[pl.ds(base, _CS), :], c_v[b],
                                  smg_c[b]).wait()
            z = jnp.zeros((16,), _f32)

            @plsc.parallel_loop(0, _CS, unroll=2, carry=(z,) * 16)
            def cr(jj, st):
                st = list(st)
                for kk in range(_D // 16):
                    sl = pl.ds(kk * 16, 16)
                    en = c_v[b][jj, sl] + a_v[b][jj, sl] + b_v[b][jj, sl]
                    sg = 1.0 / (1.0 + jnp.exp(-en))
                    c_v[b][jj, sl] = en
                    d_v[b][jj, sl] = sg
                    st[kk] = st[kk] + en
                    st[8 + kk] = st[8 + kk] + en * en
                return tuple(st)

            for kk in range(_D // 16):
                sl = pl.ds(kk * 16, 16)
                stats_b[0, sl] = stats_b[0, sl] + cr[kk]
                stats_b[1, sl] = stats_b[1, sl] + cr[8 + kk]
            if write_enew:
                pltpu.async_copy(c_v[b], enew_hbm.at[pl.ds(base, _CS), :],
                                 smw_en[b])
            pltpu.async_copy(d_v[b], sg_hbm.at[pl.ds(base, _CS), :],
                             smw_sg[b])
            pltpu.async_copy(d_v[b], acc_sh.at[dst_i[q]], smw_sc[b],
                             add=True)

        fire_idx(0, 0)
        fire_idx(1, 1)
        wait_idx(0, 0)
        fire_gathers(0, 0, 0)

        def group(i, carry):
            for q in range(4):
                jv = i * 4 + q
                b = q % 2
                fire_idx(jv + 2, (q + 2) % 4)
                wait_writes(jv - 1, 1 - b, (q + 3) % 4)
                wait_idx(jv + 1, (q + 1) % 4)
                fire_gathers(jv + 1, (q + 1) % 4, 1 - b)
                work(jv, q, b)
            return carry

        lax.fori_loop(0, chunks // 4, group, 0)
        wait_writes(chunks - 1, 1, 3)

        pltpu.sync_copy(stats_b, stats_hbm.at[s])

        plsc.subcore_barrier()
        rbase = s * _RPT
        pltpu.sync_copy(acc_sh.at[pl.ds(rbase, _RPT), :],
                        acc_hbm.at[pl.ds(rbase, _RPT), :])

    return k(ce, dh, eh, src, dst)


def _sc_pass_ssh(sg, bh, src, dst):
    per_tile = _E // _NS
    chunks = per_tile // _CS
    mesh = plsc.VectorSubcoreMesh(core_axis_name="c", subcore_axis_name="s",
                                  num_cores=1)

    out_type = [jax.ShapeDtypeStruct((_NP, _D), _f32)]
    scratch_types = (
        [pltpu.VMEM((_CS,), jnp.int32)] * 8
        + [pltpu.VMEM((_CS, _D), _f32)] * 4
        + [pltpu.VMEM_SHARED((_NP, _D), _f32)]
        + [pltpu.SemaphoreType.DMA] * 10
    )

    @functools.partial(pl.kernel, out_type=out_type, mesh=mesh,
                       scratch_types=scratch_types)
    def k(sg_hbm, bh_hbm, src_hbm, dst_hbm, acc_hbm,
          si0, si1, si2, si3, di0, di1, di2, di3,
          sg_v0, sg_v1, bh_v0, bh_v1, acc_sh, *sems):
        s = lax.axis_index("s")
        src_i = (si0, si1, si2, si3)
        dst_i = (di0, di1, di2, di3)
        sg_v = (sg_v0, sg_v1)
        bh_v = (bh_v0, bh_v1)
        sm_ix = sems[0:4]
        smg = sems[4:6]
        sml = sems[6:8]
        smw = sems[8:10]

        def zrow(j, carry):
            for kk in range(_D // 16):
                sg_v0[j, pl.ds(kk * 16, 16)] = jnp.zeros((16,), _f32)
            return carry
        lax.fori_loop(0, _CS, zrow, 0)
        for j in range(_RPT // _CS):
            off = s * _RPT + j * _CS
            pltpu.sync_copy(sg_v0, acc_sh.at[pl.ds(off, _CS), :])
        plsc.subcore_barrier()

        tbase = s * per_tile

        def fire_idx(j, q):
            @pl.when(j < chunks)
            def _():
                base = tbase + j * _CS
                pltpu.async_copy(src_hbm.at[pl.ds(base, _CS)], src_i[q],
                                 sm_ix[q])
                pltpu.async_copy(dst_hbm.at[pl.ds(base, _CS)], dst_i[q],
                                 sm_ix[q])

        def wait_idx(j, q):
            @pl.when(j < chunks)
            def _():
                pltpu.make_async_copy(src_hbm.at[pl.ds(tbase, _CS)],
                                      src_i[q], sm_ix[q]).wait()
                pltpu.make_async_copy(dst_hbm.at[pl.ds(tbase, _CS)],
                                      dst_i[q], sm_ix[q]).wait()

        def fire_gathers(j, q, b):
            @pl.when(j < chunks)
            def _():
                base = tbase + j * _CS
                pltpu.async_copy(bh_hbm.at[src_i[q]], bh_v[b], smg[b])
                pltpu.async_copy(sg_hbm.at[pl.ds(base, _CS), :], sg_v[b],
                                 sml[b])

        def wait_writes(j, b, q):
            @pl.when(j >= 0)
            def _():
                pltpu.make_async_copy(sg_v[b], acc_sh.at[dst_i[q]],
                                      smw[b]).wait()

        def work(j, q, b):
            base = tbase + j * _CS
            pltpu.make_async_copy(bh_hbm.at[src_i[q]], bh_v[b], smg[b]).wait()
            pltpu.make_async_copy(sg_hbm.at[pl.ds(base, _CS), :], sg_v[b],
                                  sml[b]).wait()

            @plsc.parallel_loop(0, _CS, unroll=4)
            def _(jj):
                for kk in range(_D // 16):
                    sl = pl.ds(kk * 16, 16)
                    sg_v[b][jj, sl] = sg_v[b][jj, sl] * bh_v[b][jj, sl]

            pltpu.async_copy(sg_v[b], acc_sh.at[dst_i[q]], smw[b], add=True)

        fire_idx(0, 0)
        fire_idx(1, 1)
        wait_idx(0, 0)
        fire_gathers(0, 0, 0)

        def group(i, carry):
            for q in range(4):
                jv = i * 4 + q
                b = q % 2
                fire_idx(jv + 2, (q + 2) % 4)
                wait_writes(jv - 1, 1 - b, (q + 3) % 4)
                wait_idx(jv + 1, (q + 1) % 4)
                fire_gathers(jv + 1, (q + 1) % 4, 1 - b)
                work(jv, q, b)
            return carry

        lax.fori_loop(0, chunks // 4, group, 0)
        wait_writes(chunks - 1, 1, 3)

        plsc.subcore_barrier()
        rbase = s * _RPT
        pltpu.sync_copy(acc_sh.at[pl.ds(rbase, _RPT), :],
                        acc_hbm.at[pl.ds(rbase, _RPT), :])

    return k(sg, bh, src, dst)


def kernel(h, p, e, snorm_n, edge_index, Wh, bh, We, be, Wp, bp, WA, bA, WB,
           bB, WC, bC, WD, bD, WEm, bEm, gh, bth, ge, bte, W1, b1, W2, b2,
           W3, b3):
    del snorm_n
    src = edge_index[0]
    dst = edge_index[1]
    L = WA.shape[0]

    hs = _input_proj(h, p, Wh, bh, Wp, bp)
    e_in = None
    enew_prev = None
    parts_prev = None
    for l in range(L):
        if l == 0:
            e_in, ce = _edge_first(e, We, be, WC[0], bC[0])
        else:
            e_in, ce = _edge_apply(e_in, enew_prev, parts_prev,
                                   ge[l - 1], bte[l - 1], WC[l], bC[l])
        Ah, Bh, Dh, Eh = _node_proj(hs, WA[l], bA[l], WB[l], bB[l],
                                    WD[l], bD[l], WEm[l], bEm[l])
        last = l == L - 1
        enew_prev, sg, ss, parts_prev = _sc_pass_ss(ce, Dh, Eh, src, dst,
                                                    write_enew=not last)
        (ssh,) = _sc_pass_ssh(sg, Bh, src, dst)
        hs = _h_update(Ah, ss[:_N], ssh[:_N], hs, gh[l], bth[l])

    y = _readout(hs, W1, b1, W2, b2, W3, b3)
    return y.reshape(10)
